# baseline (speedup 1.0000x reference)
import functools

import jax
import jax.numpy as jnp
from jax import lax
from jax.experimental import layout as jlayout
from jax.experimental import pallas as pl
from jax.experimental.pallas import tpu as pltpu
from jax.experimental.pallas import tpu_sc as plsc

_N = 50000
_B = 16
_DIM_PE = 32
_NT = 21
_DIM_IN = 128
_DIM_OUT = _DIM_IN + _DIM_PE

_PITCH = 161
_L = 1568
_CH = 112
_GC = _CH // 16
_NPAIR = _L // (2 * _CH)
_STRIDE = 1563
_LAST_START = _N - _L


def _body(t2_hbm, types_hbm, x_hbm, out_hbm,
          t2_v, types_v, rb0, rb1,
          semt, sx0, sx1, so0, so1):
    cid = lax.axis_index("c")
    sid = lax.axis_index("s")
    wid = sid * 2 + cid
    start = pl.multiple_of(jnp.minimum((wid * _STRIDE) & -16, _LAST_START), 16)

    cp_t2 = pltpu.async_copy(t2_hbm, t2_v, semt)
    cp_ty = pltpu.async_copy(types_hbm.at[:, pl.ds(start, _L)], types_v,
                             semt)
    cp_t2.wait()
    cp_ty.wait()

    viota = lax.iota(jnp.int32, 16)

    def make_compute(c, rb):
        def compute_group(j, carry):
            col16 = c * _CH + j * 16
            accs = [jnp.zeros((16,), jnp.float32) for _ in range(_DIM_PE)]
            idx0s = []
            for p in range(_B // 2):
                va = types_v[2 * p, pl.ds(col16, 16)]
                vb = types_v[2 * p + 1, pl.ds(col16, 16)]
                idx0s.append(va * _NT + vb)
            for p in range(_B // 2):
                for d in range(_DIM_PE):
                    accs[d] = accs[d] + plsc.load_gather(
                        t2_v, [idx0s[p] + d * (_NT * _NT)])
            row = j * 16 + viota
            for d in range(_DIM_PE):
                col = jnp.full((16,), _DIM_IN + d, jnp.int32)
                plsc.store_scatter(rb, [row, col], accs[d])
            return carry
        return compute_group

    def do_chunk(c, rb, sx, so, first):
        rows = pl.multiple_of(start + c * _CH, 16)

        @pl.when(jnp.logical_not(first))
        def _():
            pltpu.make_async_copy(
                rb.at[:, pl.ds(0, _DIM_OUT)],
                out_hbm.at[pl.ds(rows, _CH), :], so).wait()

        cp_in = pltpu.async_copy(
            x_hbm.at[pl.ds(rows, _CH), :], rb.at[:, pl.ds(0, _DIM_IN)], sx)
        lax.fori_loop(0, _GC, make_compute(c, rb), 0)
        cp_in.wait()
        pltpu.async_copy(
            rb.at[:, pl.ds(0, _DIM_OUT)], out_hbm.at[pl.ds(rows, _CH), :], so)

    def pair(i, carry):
        do_chunk(2 * i, rb0, sx0, so0, i == 0)
        do_chunk(2 * i + 1, rb1, sx1, so1, i == 0)
        return carry

    lax.fori_loop(0, _NPAIR, pair, 0)

    for rb, so in ((rb0, so0), (rb1, so1)):
        pltpu.make_async_copy(
            rb.at[:, pl.ds(0, _DIM_OUT)],
            out_hbm.at[pl.ds(start, _CH), :], so).wait()


def _impl(x, spatial_types, spatial_table):
    t2 = jnp.transpose(
        spatial_table[:, None, :] + spatial_table[None, :, :],
        (2, 0, 1)).reshape(_DIM_PE * _NT * _NT)

    mesh = plsc.VectorSubcoreMesh(core_axis_name="c", subcore_axis_name="s")
    out = pl.kernel(
        _body,
        out_type=jax.ShapeDtypeStruct((_N, _DIM_OUT), jnp.float32),
        mesh=mesh,
        scratch_types=[
            pltpu.VMEM((_DIM_PE * _NT * _NT,), jnp.float32),
            pltpu.VMEM((_B, _L), jnp.int32),
            pltpu.VMEM((_CH, _PITCH), jnp.float32),
            pltpu.VMEM((_CH, _PITCH), jnp.float32),
            pltpu.SemaphoreType.DMA,
            pltpu.SemaphoreType.DMA,
            pltpu.SemaphoreType.DMA,
            pltpu.SemaphoreType.DMA,
            pltpu.SemaphoreType.DMA,
        ],
        compiler_params=pltpu.CompilerParams(
            use_tc_tiling_on_sc=False, needs_layout_passes=False),
        name="hetero_pe_sc",
    )(t2, spatial_types, x)

    return jlayout.with_layout_constraint(
        out, jlayout.Layout(major_to_minor=(0, 1), tiling=()))


kernel = jax.jit(_impl)


# speedup vs baseline: 2.6934x; 2.6934x over previous
import jax
import jax.numpy as jnp
from jax import lax
from jax.experimental import pallas as pl
from jax.experimental.pallas import tpu as pltpu
from jax.experimental.pallas import tpu_sc as plsc

_N = 50000
_B = 16
_DIM_PE = 32
_DP = _DIM_PE // 2
_NT = 21
_NT2 = _NT * _NT

_PITCH = 33
_L = 1568
_G = _L // 16
_STRIDE = 1563
_LAST_START = _N - _L

_MASK_HI = jnp.int32(-65536)


def _pe_body(t2_hbm, types_hbm, out_hbm, t2_v, types_v, pe_v, sem):
    cid = lax.axis_index("c")
    sid = lax.axis_index("s")
    wid = sid * 2 + cid
    start = pl.multiple_of(jnp.minimum((wid * _STRIDE) & -16, _LAST_START), 16)

    cp_t2 = pltpu.async_copy(t2_hbm, t2_v, sem)
    cp_ty = pltpu.async_copy(types_hbm.at[:, pl.ds(start, _L)], types_v, sem)
    cp_t2.wait()
    cp_ty.wait()

    viota = lax.iota(jnp.int32, 16)

    def group(g, carry):
        base16 = g * 16
        accs = [jnp.zeros((16,), jnp.float32) for _ in range(_DIM_PE)]
        idx0s = []
        for p in range(_B // 2):
            va = types_v[2 * p, pl.ds(base16, 16)]
            vb = types_v[2 * p + 1, pl.ds(base16, 16)]
            idx0s.append(va * _NT + vb)
        for p in range(_B // 2):
            for k in range(_DP):
                w = plsc.load_gather(t2_v, [idx0s[p] + k * _NT2])
                lo = plsc.bitcast(w << 16, jnp.float32)
                hi = plsc.bitcast(w & _MASK_HI, jnp.float32)
                accs[2 * k] = accs[2 * k] + lo
                accs[2 * k + 1] = accs[2 * k + 1] + hi
        row = base16 + viota
        for d in range(_DIM_PE):
            col = jnp.full((16,), d, jnp.int32)
            plsc.store_scatter(pe_v, [row, col], accs[d])
        return carry

    lax.fori_loop(0, _G, group, 0)

    pltpu.sync_copy(
        pe_v.at[:, pl.ds(0, _DIM_PE)], out_hbm.at[pl.ds(start, _L), :])


@jax.jit
def kernel(x, spatial_types, spatial_table):
    t2 = (spatial_table[:, None, :] + spatial_table[None, :, :]).reshape(
        _NT2, _DIM_PE)
    t2b = t2.astype(jnp.bfloat16)
    bits = lax.bitcast_convert_type(t2b, jnp.uint16).astype(jnp.uint32)
    packed = bits[:, 0::2] | (bits[:, 1::2] << 16)
    packed = jnp.transpose(packed, (1, 0)).reshape(_DP * _NT2)
    packed = lax.bitcast_convert_type(packed, jnp.int32)

    mesh = plsc.VectorSubcoreMesh(core_axis_name="c", subcore_axis_name="s")
    pe = pl.kernel(
        _pe_body,
        out_type=jax.ShapeDtypeStruct((_N, _DIM_PE), jnp.float32),
        mesh=mesh,
        scratch_types=[
            pltpu.VMEM((_DP * _NT2,), jnp.int32),
            pltpu.VMEM((_B, _L), jnp.int32),
            pltpu.VMEM((_L, _PITCH), jnp.float32),
            pltpu.SemaphoreType.DMA,
        ],
        compiler_params=pltpu.CompilerParams(
            use_tc_tiling_on_sc=False, needs_layout_passes=False),
        name="hetero_pe_sc",
    )(packed, spatial_types)

    return jnp.concatenate([x, pe], axis=1)


# speedup vs baseline: 2.8407x; 1.0547x over previous
import jax
import jax.numpy as jnp
from jax import lax
from jax.experimental import pallas as pl
from jax.experimental.pallas import tpu as pltpu
from jax.experimental.pallas import tpu_sc as plsc

_N = 50000
_B = 16
_DIM_PE = 32
_DP = _DIM_PE // 2
_NT = 21
_DIM_IN = 128
_NT2 = _NT * _NT

_PITCH = 33
_L = 1568
_G = _L // 16
_STRIDE = 1563
_LAST_START = _N - _L

_MASK_HI = jnp.int32(-65536)


def _pe_body(t2_hbm, types_hbm, out_hbm, t2_v, types_v, pe_v, sem):
    cid = lax.axis_index("c")
    sid = lax.axis_index("s")
    wid = sid * 2 + cid
    start = pl.multiple_of(jnp.minimum((wid * _STRIDE) & -16, _LAST_START), 16)

    cp_t2 = pltpu.async_copy(t2_hbm, t2_v, sem)
    cp_ty = pltpu.async_copy(types_hbm.at[:, pl.ds(start, _L)], types_v, sem)
    cp_t2.wait()
    cp_ty.wait()

    viota = lax.iota(jnp.int32, 16)

    def group(g, carry):
        base16 = g * 16
        accs = [jnp.zeros((16,), jnp.float32) for _ in range(_DIM_PE)]
        idx0s = []
        for p in range(_B // 2):
            va = types_v[2 * p, pl.ds(base16, 16)]
            vb = types_v[2 * p + 1, pl.ds(base16, 16)]
            idx0s.append(va * _NT + vb)
        for p in range(_B // 2):
            for k in range(_DP):
                w = plsc.load_gather(t2_v, [idx0s[p] + k * _NT2])
                lo = plsc.bitcast(w << 16, jnp.float32)
                hi = plsc.bitcast(w & _MASK_HI, jnp.float32)
                accs[2 * k] = accs[2 * k] + lo
                accs[2 * k + 1] = accs[2 * k + 1] + hi
        row = base16 + viota
        for d in range(_DIM_PE):
            col = jnp.full((16,), d, jnp.int32)
            plsc.store_scatter(pe_v, [row, col], accs[d])
        return carry

    lax.fori_loop(0, _G, group, 0)

    pltpu.sync_copy(
        pe_v.at[:, pl.ds(0, _DIM_PE)], out_hbm.at[pl.ds(start, _L), :])


@jax.jit
def kernel(x, spatial_types, spatial_table):
    t2 = (spatial_table[:, None, :] + spatial_table[None, :, :]).reshape(
        _NT2, _DIM_PE)
    t2b = t2.astype(jnp.bfloat16)
    bits = lax.bitcast_convert_type(t2b, jnp.uint16).astype(jnp.uint32)
    packed = bits[:, 0::2] | (bits[:, 1::2] << 16)
    packed = jnp.transpose(packed, (1, 0)).reshape(_DP * _NT2)
    packed = lax.bitcast_convert_type(packed, jnp.int32)

    mesh = plsc.VectorSubcoreMesh(core_axis_name="c", subcore_axis_name="s")
    pe = pl.kernel(
        _pe_body,
        out_type=jax.ShapeDtypeStruct((_N, _DIM_PE), jnp.float32),
        mesh=mesh,
        scratch_types=[
            pltpu.VMEM((_DP * _NT2,), jnp.int32),
            pltpu.VMEM((_B, _L), jnp.int32),
            pltpu.VMEM((_L, _PITCH), jnp.float32),
            pltpu.SemaphoreType.DMA,
        ],
        compiler_params=pltpu.CompilerParams(
            use_tc_tiling_on_sc=False, needs_layout_passes=False),
        name="hetero_pe_sc",
    )(packed, spatial_types)

    out0 = jnp.pad(x, ((0, 0), (0, _DIM_PE)))
    return lax.dynamic_update_slice(out0, pe, (0, _DIM_IN))
